# Initial kernel scaffold; baseline (speedup 1.0000x reference)
import jax
import jax.numpy as jnp
from jax.experimental import pallas as pl


def kernel(xyz1, xyz2):
    raise NotImplementedError("write your pallas kernel here")


# speedup vs baseline: 1.5623x; 1.5623x over previous
import functools

import jax
import jax.numpy as jnp
from jax.experimental import pallas as pl
from jax.experimental.pallas import tpu as pltpu

_BN = 256


def _chamfer_body(x1_ref, x2t_ref, d1_ref, i1_ref, d2_ref, i2_ref, *, bn, m):
    ib = pl.program_id(1)

    x1 = x1_ref[0]
    x2t = x2t_ref[0]

    sq1 = jnp.sum(x1 * x1, axis=1, keepdims=True)
    sq2 = jnp.sum(x2t * x2t, axis=0, keepdims=True)

    x1b = x1.astype(jnp.bfloat16).astype(jnp.float32)
    x2b = x2t.astype(jnp.bfloat16).astype(jnp.float32)
    inner = (x1b[:, 0:1] * x2b[0:1, :]
             + x1b[:, 1:2] * x2b[1:2, :]
             + x1b[:, 2:3] * x2b[2:3, :])
    d = jnp.maximum(sq1 + sq2 - 2.0 * inner, 0.0)

    big = jnp.int32(2 ** 30)

    rmin = jnp.min(d, axis=1, keepdims=True)
    jcol = jax.lax.broadcasted_iota(jnp.int32, (bn, m), 1)
    rarg = jnp.min(jnp.where(d == rmin, jcol, big), axis=1, keepdims=True)
    d1_ref[0] = rmin
    i1_ref[0] = rarg

    cmin = jnp.min(d, axis=0, keepdims=True)
    irow = jax.lax.broadcasted_iota(jnp.int32, (bn, m), 0) + ib * bn
    carg = jnp.min(jnp.where(d == cmin, irow, big), axis=0, keepdims=True)

    @pl.when(ib == 0)
    def _init():
        d2_ref[0] = cmin
        i2_ref[0] = carg

    @pl.when(ib > 0)
    def _update():
        prev_d = d2_ref[0]
        prev_i = i2_ref[0]
        better = cmin < prev_d
        d2_ref[0] = jnp.where(better, cmin, prev_d)
        i2_ref[0] = jnp.where(better, carg, prev_i)


@jax.jit
def kernel(xyz1, xyz2):
    B, n, _ = xyz1.shape
    m = xyz2.shape[1]
    bn = _BN
    nb = n // bn

    x2t = jnp.transpose(xyz2, (0, 2, 1))

    grid = (B, nb)
    out_shapes = (
        jax.ShapeDtypeStruct((B, n, 1), jnp.float32),
        jax.ShapeDtypeStruct((B, n, 1), jnp.int32),
        jax.ShapeDtypeStruct((B, 1, m), jnp.float32),
        jax.ShapeDtypeStruct((B, 1, m), jnp.int32),
    )
    in_specs = [
        pl.BlockSpec((1, bn, 3), lambda b, ib: (b, ib, 0)),
        pl.BlockSpec((1, 3, m), lambda b, ib: (b, 0, 0)),
    ]
    out_specs = (
        pl.BlockSpec((1, bn, 1), lambda b, ib: (b, ib, 0)),
        pl.BlockSpec((1, bn, 1), lambda b, ib: (b, ib, 0)),
        pl.BlockSpec((1, 1, m), lambda b, ib: (b, 0, 0)),
        pl.BlockSpec((1, 1, m), lambda b, ib: (b, 0, 0)),
    )

    d1, i1, d2, i2 = pl.pallas_call(
        functools.partial(_chamfer_body, bn=bn, m=m),
        grid=grid,
        in_specs=in_specs,
        out_specs=out_specs,
        out_shape=out_shapes,
        compiler_params=pltpu.CompilerParams(
            dimension_semantics=("parallel", "arbitrary"),
        ),
    )(xyz1, x2t)

    return (d1[:, :, 0], d2[:, 0, :], i1[:, :, 0], i2[:, 0, :])


# speedup vs baseline: 1.8802x; 1.2035x over previous
import functools

import jax
import jax.numpy as jnp
from jax.experimental import pallas as pl
from jax.experimental.pallas import tpu as pltpu

_BN = 256


def _chamfer_body(x1_ref, x2t_ref, d1_ref, i1_ref, d2_ref, i2_ref, *, bn, m):
    ib = pl.program_id(1)

    x1 = x1_ref[0]
    x2t = x2t_ref[0]

    sq1 = jnp.sum(x1 * x1, axis=1, keepdims=True)
    sq2 = jnp.sum(x2t * x2t, axis=0, keepdims=True)

    inner = jnp.dot(x1, x2t, preferred_element_type=jnp.float32)
    d = jnp.maximum(sq1 + sq2 - 2.0 * inner, 0.0)

    big = jnp.int32(2 ** 30)

    rmin = jnp.min(d, axis=1, keepdims=True)
    jcol = jax.lax.broadcasted_iota(jnp.int32, (bn, m), 1)
    rarg = jnp.min(jnp.where(d == rmin, jcol, big), axis=1, keepdims=True)
    d1_ref[0] = rmin
    i1_ref[0] = rarg

    cmin = jnp.min(d, axis=0, keepdims=True)
    irow = jax.lax.broadcasted_iota(jnp.int32, (bn, m), 0) + ib * bn
    carg = jnp.min(jnp.where(d == cmin, irow, big), axis=0, keepdims=True)

    @pl.when(ib == 0)
    def _init():
        d2_ref[0] = cmin
        i2_ref[0] = carg

    @pl.when(ib > 0)
    def _update():
        prev_d = d2_ref[0]
        prev_i = i2_ref[0]
        better = cmin < prev_d
        d2_ref[0] = jnp.where(better, cmin, prev_d)
        i2_ref[0] = jnp.where(better, carg, prev_i)


@jax.jit
def kernel(xyz1, xyz2):
    B, n, _ = xyz1.shape
    m = xyz2.shape[1]
    bn = _BN
    nb = n // bn

    x1p = jnp.pad(xyz1, ((0, 0), (0, 0), (0, 5)))
    x2t = jnp.pad(jnp.transpose(xyz2, (0, 2, 1)),
                  ((0, 0), (0, 5), (0, 0)))

    grid = (B, nb)
    out_shapes = (
        jax.ShapeDtypeStruct((B, n, 1), jnp.float32),
        jax.ShapeDtypeStruct((B, n, 1), jnp.int32),
        jax.ShapeDtypeStruct((B, 1, m), jnp.float32),
        jax.ShapeDtypeStruct((B, 1, m), jnp.int32),
    )
    in_specs = [
        pl.BlockSpec((1, bn, 8), lambda b, ib: (b, ib, 0)),
        pl.BlockSpec((1, 8, m), lambda b, ib: (b, 0, 0)),
    ]
    out_specs = (
        pl.BlockSpec((1, bn, 1), lambda b, ib: (b, ib, 0)),
        pl.BlockSpec((1, bn, 1), lambda b, ib: (b, ib, 0)),
        pl.BlockSpec((1, 1, m), lambda b, ib: (b, 0, 0)),
        pl.BlockSpec((1, 1, m), lambda b, ib: (b, 0, 0)),
    )

    d1, i1, d2, i2 = pl.pallas_call(
        functools.partial(_chamfer_body, bn=bn, m=m),
        grid=grid,
        in_specs=in_specs,
        out_specs=out_specs,
        out_shape=out_shapes,
        compiler_params=pltpu.CompilerParams(
            dimension_semantics=("parallel", "arbitrary"),
        ),
    )(x1p, x2t)

    return (d1[:, :, 0], d2[:, 0, :], i1[:, :, 0], i2[:, 0, :])


# speedup vs baseline: 1.9822x; 1.0542x over previous
import functools

import jax
import jax.numpy as jnp
from jax.experimental import pallas as pl
from jax.experimental.pallas import tpu as pltpu

_BN = 512


def _chamfer_body(x1_ref, x2t_ref, d1_ref, i1_ref, d2_ref, i2_ref, *, bn, m):
    ib = pl.program_id(1)

    x1 = x1_ref[0]
    x2t = x2t_ref[0]

    sq1 = jnp.sum(x1 * x1, axis=1, keepdims=True)
    sq2 = jnp.sum(x2t * x2t, axis=0, keepdims=True)

    inner = jnp.dot(x1, x2t, preferred_element_type=jnp.float32)
    d = jnp.maximum(sq1 + sq2 - 2.0 * inner, 0.0)

    big = jnp.int32(2 ** 30)

    rmin = jnp.min(d, axis=1, keepdims=True)
    jcol = jax.lax.broadcasted_iota(jnp.int32, (bn, m), 1)
    rarg = jnp.min(jnp.where(d == rmin, jcol, big), axis=1, keepdims=True)
    d1_ref[0] = rmin
    i1_ref[0] = rarg

    cmin = jnp.min(d, axis=0, keepdims=True)
    irow = jax.lax.broadcasted_iota(jnp.int32, (bn, m), 0) + ib * bn
    carg = jnp.min(jnp.where(d == cmin, irow, big), axis=0, keepdims=True)

    @pl.when(ib == 0)
    def _init():
        d2_ref[0] = cmin
        i2_ref[0] = carg

    @pl.when(ib > 0)
    def _update():
        prev_d = d2_ref[0]
        prev_i = i2_ref[0]
        better = cmin < prev_d
        d2_ref[0] = jnp.where(better, cmin, prev_d)
        i2_ref[0] = jnp.where(better, carg, prev_i)


@jax.jit
def kernel(xyz1, xyz2):
    B, n, _ = xyz1.shape
    m = xyz2.shape[1]
    bn = _BN
    nb = n // bn

    x1p = jnp.pad(xyz1, ((0, 0), (0, 0), (0, 5)))
    x2t = jnp.pad(jnp.transpose(xyz2, (0, 2, 1)),
                  ((0, 0), (0, 5), (0, 0)))

    grid = (B, nb)
    out_shapes = (
        jax.ShapeDtypeStruct((B, n, 1), jnp.float32),
        jax.ShapeDtypeStruct((B, n, 1), jnp.int32),
        jax.ShapeDtypeStruct((B, 1, m), jnp.float32),
        jax.ShapeDtypeStruct((B, 1, m), jnp.int32),
    )
    in_specs = [
        pl.BlockSpec((1, bn, 8), lambda b, ib: (b, ib, 0)),
        pl.BlockSpec((1, 8, m), lambda b, ib: (b, 0, 0)),
    ]
    out_specs = (
        pl.BlockSpec((1, bn, 1), lambda b, ib: (b, ib, 0)),
        pl.BlockSpec((1, bn, 1), lambda b, ib: (b, ib, 0)),
        pl.BlockSpec((1, 1, m), lambda b, ib: (b, 0, 0)),
        pl.BlockSpec((1, 1, m), lambda b, ib: (b, 0, 0)),
    )

    d1, i1, d2, i2 = pl.pallas_call(
        functools.partial(_chamfer_body, bn=bn, m=m),
        grid=grid,
        in_specs=in_specs,
        out_specs=out_specs,
        out_shape=out_shapes,
        compiler_params=pltpu.CompilerParams(
            dimension_semantics=("parallel", "arbitrary"),
        ),
    )(x1p, x2t)

    return (d1[:, :, 0], d2[:, 0, :], i1[:, :, 0], i2[:, 0, :])


# speedup vs baseline: 2.0693x; 1.0439x over previous
import functools

import jax
import jax.numpy as jnp
from jax.experimental import pallas as pl
from jax.experimental.pallas import tpu as pltpu

_BN = 1024


def _chamfer_body(x1_ref, x2t_ref, d1_ref, i1_ref, d2_ref, i2_ref, *, bn, m):
    ib = pl.program_id(1)

    x1 = x1_ref[0]
    x2t = x2t_ref[0]

    sq1 = jnp.sum(x1 * x1, axis=1, keepdims=True)
    sq2 = jnp.sum(x2t * x2t, axis=0, keepdims=True)

    inner = jnp.dot(x1, x2t, preferred_element_type=jnp.float32)
    d = jnp.maximum(sq1 + sq2 - 2.0 * inner, 0.0)

    big = jnp.int32(2 ** 30)

    rmin = jnp.min(d, axis=1, keepdims=True)
    jcol = jax.lax.broadcasted_iota(jnp.int32, (bn, m), 1)
    rarg = jnp.min(jnp.where(d == rmin, jcol, big), axis=1, keepdims=True)
    d1_ref[0] = rmin
    i1_ref[0] = rarg

    cmin = jnp.min(d, axis=0, keepdims=True)
    irow = jax.lax.broadcasted_iota(jnp.int32, (bn, m), 0) + ib * bn
    carg = jnp.min(jnp.where(d == cmin, irow, big), axis=0, keepdims=True)

    @pl.when(ib == 0)
    def _init():
        d2_ref[0] = cmin
        i2_ref[0] = carg

    @pl.when(ib > 0)
    def _update():
        prev_d = d2_ref[0]
        prev_i = i2_ref[0]
        better = cmin < prev_d
        d2_ref[0] = jnp.where(better, cmin, prev_d)
        i2_ref[0] = jnp.where(better, carg, prev_i)


@jax.jit
def kernel(xyz1, xyz2):
    B, n, _ = xyz1.shape
    m = xyz2.shape[1]
    bn = _BN
    nb = n // bn

    x1p = jnp.pad(xyz1, ((0, 0), (0, 0), (0, 5)))
    x2t = jnp.pad(jnp.transpose(xyz2, (0, 2, 1)),
                  ((0, 0), (0, 5), (0, 0)))

    grid = (B, nb)
    out_shapes = (
        jax.ShapeDtypeStruct((B, n, 1), jnp.float32),
        jax.ShapeDtypeStruct((B, n, 1), jnp.int32),
        jax.ShapeDtypeStruct((B, 1, m), jnp.float32),
        jax.ShapeDtypeStruct((B, 1, m), jnp.int32),
    )
    in_specs = [
        pl.BlockSpec((1, bn, 8), lambda b, ib: (b, ib, 0)),
        pl.BlockSpec((1, 8, m), lambda b, ib: (b, 0, 0)),
    ]
    out_specs = (
        pl.BlockSpec((1, bn, 1), lambda b, ib: (b, ib, 0)),
        pl.BlockSpec((1, bn, 1), lambda b, ib: (b, ib, 0)),
        pl.BlockSpec((1, 1, m), lambda b, ib: (b, 0, 0)),
        pl.BlockSpec((1, 1, m), lambda b, ib: (b, 0, 0)),
    )

    d1, i1, d2, i2 = pl.pallas_call(
        functools.partial(_chamfer_body, bn=bn, m=m),
        grid=grid,
        in_specs=in_specs,
        out_specs=out_specs,
        out_shape=out_shapes,
        compiler_params=pltpu.CompilerParams(
            dimension_semantics=("parallel", "arbitrary"),
        ),
    )(x1p, x2t)

    return (d1[:, :, 0], d2[:, 0, :], i1[:, :, 0], i2[:, 0, :])


# speedup vs baseline: 2.2818x; 1.1027x over previous
import functools

import jax
import jax.numpy as jnp
from jax.experimental import pallas as pl
from jax.experimental.pallas import tpu as pltpu

_BN = 1024


def _chamfer_body(x1_ref, x2t_ref, d1_ref, i1_ref, d2_ref, i2_ref, *, bn, m):
    ib = pl.program_id(1)

    x1 = x1_ref[0]
    x2t = x2t_ref[0]

    sq1 = jnp.sum(x1 * x1, axis=1, keepdims=True)
    y = -0.5 * x2t
    sq2 = jnp.sum(y * y, axis=0, keepdims=True)

    inner = jnp.dot(x1, x2t, preferred_element_type=jnp.float32)
    d = jnp.maximum((sq1 + sq2) + inner, 0.0)

    big = jnp.float32(2.0 ** 30)

    rmin = jnp.min(d, axis=1, keepdims=True)
    jcol = jax.lax.broadcasted_iota(
        jnp.int32, (1, m), 1).astype(jnp.float32)
    rarg = jnp.min(jnp.where(d == rmin, jcol, big), axis=1,
                   keepdims=True).astype(jnp.int32)
    d1_ref[0] = rmin
    i1_ref[0] = rarg

    cmin = jnp.min(d, axis=0, keepdims=True)
    irow = (jax.lax.broadcasted_iota(jnp.int32, (bn, 1), 0)
            + ib * bn).astype(jnp.float32)
    carg = jnp.min(jnp.where(d == cmin, irow, big), axis=0,
                   keepdims=True).astype(jnp.int32)

    @pl.when(ib == 0)
    def _init():
        d2_ref[0] = cmin
        i2_ref[0] = carg

    @pl.when(ib > 0)
    def _update():
        prev_d = d2_ref[0]
        prev_i = i2_ref[0]
        better = cmin < prev_d
        d2_ref[0] = jnp.where(better, cmin, prev_d)
        i2_ref[0] = jnp.where(better, carg, prev_i)


@jax.jit
def kernel(xyz1, xyz2):
    B, n, _ = xyz1.shape
    m = xyz2.shape[1]
    bn = _BN
    nb = n // bn

    x1p = jnp.pad(xyz1, ((0, 0), (0, 0), (0, 5)))
    x2t = jnp.pad(jnp.transpose(-2.0 * xyz2, (0, 2, 1)),
                  ((0, 0), (0, 5), (0, 0)))

    grid = (B, nb)
    out_shapes = (
        jax.ShapeDtypeStruct((B, n, 1), jnp.float32),
        jax.ShapeDtypeStruct((B, n, 1), jnp.int32),
        jax.ShapeDtypeStruct((B, 1, m), jnp.float32),
        jax.ShapeDtypeStruct((B, 1, m), jnp.int32),
    )
    in_specs = [
        pl.BlockSpec((1, bn, 8), lambda b, ib: (b, ib, 0)),
        pl.BlockSpec((1, 8, m), lambda b, ib: (b, 0, 0)),
    ]
    out_specs = (
        pl.BlockSpec((1, bn, 1), lambda b, ib: (b, ib, 0)),
        pl.BlockSpec((1, bn, 1), lambda b, ib: (b, ib, 0)),
        pl.BlockSpec((1, 1, m), lambda b, ib: (b, 0, 0)),
        pl.BlockSpec((1, 1, m), lambda b, ib: (b, 0, 0)),
    )

    d1, i1, d2, i2 = pl.pallas_call(
        functools.partial(_chamfer_body, bn=bn, m=m),
        grid=grid,
        in_specs=in_specs,
        out_specs=out_specs,
        out_shape=out_shapes,
        compiler_params=pltpu.CompilerParams(
            dimension_semantics=("parallel", "arbitrary"),
        ),
    )(x1p, x2t)

    return (d1[:, :, 0], d2[:, 0, :], i1[:, :, 0], i2[:, 0, :])


# speedup vs baseline: 2.4239x; 1.0622x over previous
import functools

import jax
import jax.numpy as jnp
from jax.experimental import pallas as pl
from jax.experimental.pallas import tpu as pltpu

_BN = 2048


def _chamfer_body(x1_ref, x2t_ref, d1_ref, i1_ref, d2_ref, i2_ref, *, bn, m):
    ib = pl.program_id(1)

    x1 = x1_ref[0]
    x2t = x2t_ref[0]

    sq1 = jnp.sum(x1 * x1, axis=1, keepdims=True)
    y = -0.5 * x2t
    sq2 = jnp.sum(y * y, axis=0, keepdims=True)

    inner = jnp.dot(x1, x2t, preferred_element_type=jnp.float32)
    d = jnp.maximum((sq1 + sq2) + inner, 0.0)

    big = jnp.float32(2.0 ** 30)

    rmin = jnp.min(d, axis=1, keepdims=True)
    jcol = jax.lax.broadcasted_iota(
        jnp.int32, (1, m), 1).astype(jnp.float32)
    rarg = jnp.min(jnp.where(d == rmin, jcol, big), axis=1,
                   keepdims=True).astype(jnp.int32)
    d1_ref[0] = rmin
    i1_ref[0] = rarg

    cmin = jnp.min(d, axis=0, keepdims=True)
    irow = (jax.lax.broadcasted_iota(jnp.int32, (bn, 1), 0)
            + ib * bn).astype(jnp.float32)
    carg = jnp.min(jnp.where(d == cmin, irow, big), axis=0,
                   keepdims=True).astype(jnp.int32)

    @pl.when(ib == 0)
    def _init():
        d2_ref[0] = cmin
        i2_ref[0] = carg

    @pl.when(ib > 0)
    def _update():
        prev_d = d2_ref[0]
        prev_i = i2_ref[0]
        better = cmin < prev_d
        d2_ref[0] = jnp.where(better, cmin, prev_d)
        i2_ref[0] = jnp.where(better, carg, prev_i)


@jax.jit
def kernel(xyz1, xyz2):
    B, n, _ = xyz1.shape
    m = xyz2.shape[1]
    bn = _BN
    nb = n // bn

    x1p = jnp.pad(xyz1, ((0, 0), (0, 0), (0, 5)))
    x2t = jnp.pad(jnp.transpose(-2.0 * xyz2, (0, 2, 1)),
                  ((0, 0), (0, 5), (0, 0)))

    grid = (B, nb)
    out_shapes = (
        jax.ShapeDtypeStruct((B, n, 1), jnp.float32),
        jax.ShapeDtypeStruct((B, n, 1), jnp.int32),
        jax.ShapeDtypeStruct((B, 1, m), jnp.float32),
        jax.ShapeDtypeStruct((B, 1, m), jnp.int32),
    )
    in_specs = [
        pl.BlockSpec((1, bn, 8), lambda b, ib: (b, ib, 0)),
        pl.BlockSpec((1, 8, m), lambda b, ib: (b, 0, 0)),
    ]
    out_specs = (
        pl.BlockSpec((1, bn, 1), lambda b, ib: (b, ib, 0)),
        pl.BlockSpec((1, bn, 1), lambda b, ib: (b, ib, 0)),
        pl.BlockSpec((1, 1, m), lambda b, ib: (b, 0, 0)),
        pl.BlockSpec((1, 1, m), lambda b, ib: (b, 0, 0)),
    )

    d1, i1, d2, i2 = pl.pallas_call(
        functools.partial(_chamfer_body, bn=bn, m=m),
        grid=grid,
        in_specs=in_specs,
        out_specs=out_specs,
        out_shape=out_shapes,
        compiler_params=pltpu.CompilerParams(
            dimension_semantics=("parallel", "arbitrary"),
        ),
    )(x1p, x2t)

    return (d1[:, :, 0], d2[:, 0, :], i1[:, :, 0], i2[:, 0, :])
